# baseline (speedup 1.0000x reference)
import functools
import math

import jax
import jax.numpy as jnp
from jax import lax
from jax.experimental import pallas as pl
from jax.experimental.pallas import tpu as pltpu
from jax.experimental.pallas import tpu_sc as plsc

VOCAB = 100000
SMOOTH = 0.1 / (VOCAB - 2)
CONF = 0.9
ROW_ENT = 0.1 * math.log(SMOOTH) + CONF * math.log(CONF)

NC, NS = 2, 16
NW = NC * NS
LANES = 16

TAIL0 = (VOCAB // 128) * 128
CW = 4096
N_SC_CHUNKS = 5
WSC = CW * N_SC_CHUNKS
CS = TAIL0 - WSC

BR = 64


def _dense_body(t_ref, x_ref, xt_ref, out_ref, acc_ref):
    i = pl.program_id(0)

    @pl.when(i == 0)
    def _():
        acc_ref[0] = 0.0

    tb = t_ref[...]
    validf = (tb != 0).astype(jnp.float32)

    col = lax.broadcasted_iota(jnp.int32, (BR, CS), 1)
    coef = jnp.where(col == tb, jnp.float32(-CONF),
                     jnp.where(col == 0, jnp.float32(0.0),
                               jnp.float32(-SMOOTH)))
    part = jnp.sum(x_ref[...] * coef * validf)

    colt = lax.broadcasted_iota(jnp.int32, (BR, 128), 1) + TAIL0
    xt = jnp.where(colt < VOCAB, xt_ref[...], 0.0)
    coeft = jnp.where(colt == tb, jnp.float32(-CONF), jnp.float32(-SMOOTH))
    part += jnp.sum(xt * coeft * validf)

    part += jnp.sum(validf) * jnp.float32(ROW_ENT)
    acc_ref[0] += part

    @pl.when(i == pl.num_programs(0) - 1)
    def _():
        out_ref[0, 0] = acc_ref[0]


def _dense_sum(t2d, x, interpret=False):
    n, v = x.shape
    ni = pl.cdiv(n, BR)
    return pl.pallas_call(
        _dense_body,
        grid=(ni,),
        in_specs=[
            pl.BlockSpec((BR, 1), lambda i: (i, 0)),
            pl.BlockSpec((BR, CS), lambda i: (i, 0)),
            pl.BlockSpec((BR, 128), lambda i: (i, TAIL0 // 128)),
        ],
        out_specs=pl.BlockSpec((1, 1), lambda i: (0, 0),
                               memory_space=pltpu.SMEM),
        out_shape=jax.ShapeDtypeStruct((1, 1), jnp.float32),
        scratch_shapes=[pltpu.SMEM((1,), jnp.float32)],
        interpret=interpret,
    )(t2d, x, x)


def _build_sc_dense(n):
    rpw = n // NW
    nstripe = rpw // 8
    assert rpw % LANES == 0 and rpw % 8 == 0
    mesh = plsc.VectorSubcoreMesh(core_axis_name="c", subcore_axis_name="s")

    @functools.partial(
        pl.kernel,
        mesh=mesh,
        out_type=jax.ShapeDtypeStruct((NW, LANES), jnp.float32),
        scratch_types=[
            pltpu.VMEM((rpw,), jnp.int32),
            pltpu.VMEM((8, CW), jnp.float32),
            pltpu.VMEM((8, CW), jnp.float32),
            pltpu.VMEM((LANES,), jnp.float32),
            pltpu.SemaphoreType.DMA,
            pltpu.SemaphoreType.DMA,
        ],
    )
    def sc_fn(x_hbm, tgt_hbm, out_hbm, t_v, buf0, buf1, acc_v, sem0, sem1):
        wid = lax.axis_index("s") * NC + lax.axis_index("c")
        base = pl.multiple_of(wid * rpw, 8)
        pltpu.sync_copy(tgt_hbm.at[pl.ds(base, rpw)], t_v)

        lane_ids = lax.iota(jnp.int32, LANES)
        tbs = []
        for k in range(rpw):
            t_vec = t_v[pl.ds((k // LANES) * LANES, LANES)]
            tb = lax.gather(
                t_vec,
                jnp.full((LANES, 1), k % LANES, jnp.int32),
                dimension_numbers=lax.GatherDimensionNumbers(
                    offset_dims=(), collapsed_slice_dims=(0,),
                    start_index_map=(0,)),
                slice_sizes=(1,),
                mode=lax.GatherScatterMode.PROMISE_IN_BOUNDS)
            tbs.append(tb)

        bufs = (buf0, buf1)
        sems = (sem0, sem1)
        chunks = [(s, c) for s in range(nstripe) for c in range(N_SC_CHUNKS)]

        def fire(i):
            s, c = chunks[i]
            return pltpu.async_copy(
                x_hbm.at[pl.ds(base + s * 8, 8), pl.ds(CS + c * CW, CW)],
                bufs[i % 2], sems[i % 2])

        cp = fire(0)
        loss = jnp.zeros((LANES,), jnp.float32)
        for i in range(len(chunks)):
            nxt = fire(i + 1) if i + 1 < len(chunks) else None
            cp.wait()
            s, c = chunks[i]
            b = bufs[i % 2]
            c0 = CS + c * CW
            for r in range(8):
                k = s * 8 + r
                tb = tbs[k]
                validf = jnp.where(tb != 0, jnp.float32(1.0),
                                   jnp.float32(0.0))
                def body(iv, carry, b=b, r=r, tb=tb, c0=c0):
                    acc_a, acc_g = carry
                    xv = b[r, pl.ds(iv * LANES, LANES)]
                    colv = (jnp.zeros((LANES,), jnp.int32) + iv) * LANES \
                        + lane_ids
                    hitf = jnp.where(colv + c0 == tb, jnp.float32(1.0),
                                     jnp.float32(0.0))
                    return acc_a + xv, acc_g + xv * hitf

                acc_a, acc_g = lax.fori_loop(
                    0, CW // LANES, body,
                    (jnp.zeros((LANES,), jnp.float32),
                     jnp.zeros((LANES,), jnp.float32)))
                loss = loss + (jnp.float32(-SMOOTH) * acc_a
                               + jnp.float32(SMOOTH - CONF) * acc_g) * validf
            cp = nxt
        acc_v[...] = loss
        pltpu.sync_copy(acc_v, out_hbm.at[wid])

    return sc_fn


_sc_dense_cached = functools.lru_cache(maxsize=None)(_build_sc_dense)


def kernel(x, target):
    n, _ = x.shape
    t32 = target.astype(jnp.int32)
    sc_part = _sc_dense_cached(n)(x, t32)
    tc_part = _dense_sum(t32.reshape(n, 1), x)
    return tc_part[0, 0] + jnp.sum(sc_part)


# speedup vs baseline: 1.0890x; 1.0890x over previous
import functools
import math

import jax
import jax.numpy as jnp
from jax import lax
from jax.experimental import pallas as pl
from jax.experimental.pallas import tpu as pltpu
from jax.experimental.pallas import tpu_sc as plsc

VOCAB = 100000
SMOOTH = 0.1 / (VOCAB - 2)
CONF = 0.9
ROW_ENT = 0.1 * math.log(SMOOTH) + CONF * math.log(CONF)

NC, NS = 2, 16
NW = NC * NS
LANES = 16

TAIL0 = (VOCAB // 128) * 128
CW = 4096
N_SC_CHUNKS = 5
WSC = CW * N_SC_CHUNKS
CS = TAIL0 - WSC

BR = 64


def _dense_body(t_ref, x_ref, xt_ref, out_ref, acc_ref):
    i = pl.program_id(0)

    @pl.when(i == 0)
    def _():
        acc_ref[0] = 0.0

    tb = t_ref[...]
    validf = (tb != 0).astype(jnp.float32)

    col = lax.broadcasted_iota(jnp.int32, (BR, CS), 1)
    coef = jnp.where(col == tb, jnp.float32(-CONF),
                     jnp.where(col == 0, jnp.float32(0.0),
                               jnp.float32(-SMOOTH)))
    part = jnp.sum(x_ref[...] * coef * validf)

    colt = lax.broadcasted_iota(jnp.int32, (BR, 128), 1) + TAIL0
    xt = jnp.where(colt < VOCAB, xt_ref[...], 0.0)
    coeft = jnp.where(colt == tb, jnp.float32(-CONF), jnp.float32(-SMOOTH))
    part += jnp.sum(xt * coeft * validf)

    part += jnp.sum(validf) * jnp.float32(ROW_ENT)
    acc_ref[0] += part

    @pl.when(i == pl.num_programs(0) - 1)
    def _():
        out_ref[0, 0] = acc_ref[0]


def _dense_sum(t2d, x, interpret=False):
    n, v = x.shape
    ni = pl.cdiv(n, BR)
    return pl.pallas_call(
        _dense_body,
        grid=(ni,),
        in_specs=[
            pl.BlockSpec((BR, 1), lambda i: (i, 0)),
            pl.BlockSpec((BR, CS), lambda i: (i, 0)),
            pl.BlockSpec((BR, 128), lambda i: (i, TAIL0 // 128)),
        ],
        out_specs=pl.BlockSpec((1, 1), lambda i: (0, 0),
                               memory_space=pltpu.SMEM),
        out_shape=jax.ShapeDtypeStruct((1, 1), jnp.float32),
        scratch_shapes=[pltpu.SMEM((1,), jnp.float32)],
        interpret=interpret,
    )(t2d, x, x)


def _build_sc_dense(n):
    rpw = n // NW
    nstripe = rpw // 8
    assert rpw % LANES == 0 and rpw % 8 == 0
    mesh = plsc.VectorSubcoreMesh(core_axis_name="c", subcore_axis_name="s")

    @functools.partial(
        pl.kernel,
        mesh=mesh,
        out_type=jax.ShapeDtypeStruct((NW, LANES), jnp.float32),
        scratch_types=[
            pltpu.VMEM((rpw,), jnp.int32),
            pltpu.VMEM((8, CW), jnp.float32),
            pltpu.VMEM((8, CW), jnp.float32),
            pltpu.VMEM((LANES,), jnp.float32),
            pltpu.SemaphoreType.DMA,
            pltpu.SemaphoreType.DMA,
        ],
    )
    def sc_fn(x_hbm, tgt_hbm, out_hbm, t_v, buf0, buf1, acc_v, sem0, sem1):
        wid = lax.axis_index("s") * NC + lax.axis_index("c")
        base = pl.multiple_of(wid * rpw, 8)
        pltpu.sync_copy(tgt_hbm.at[pl.ds(base, rpw)], t_v)

        lane_ids = lax.iota(jnp.int32, LANES)
        tbs = []
        for k in range(rpw):
            t_vec = t_v[pl.ds((k // LANES) * LANES, LANES)]
            tb = lax.gather(
                t_vec,
                jnp.full((LANES, 1), k % LANES, jnp.int32),
                dimension_numbers=lax.GatherDimensionNumbers(
                    offset_dims=(), collapsed_slice_dims=(0,),
                    start_index_map=(0,)),
                slice_sizes=(1,),
                mode=lax.GatherScatterMode.PROMISE_IN_BOUNDS)
            tbs.append(tb)

        bufs = (buf0, buf1)
        sems = (sem0, sem1)
        chunks = [(s, c) for s in range(nstripe) for c in range(N_SC_CHUNKS)]

        def fire(i):
            s, c = chunks[i]
            return pltpu.async_copy(
                x_hbm.at[pl.ds(base + s * 8, 8), pl.ds(CS + c * CW, CW)],
                bufs[i % 2], sems[i % 2])

        cp = fire(0)
        loss = jnp.zeros((LANES,), jnp.float32)
        for i in range(len(chunks)):
            nxt = fire(i + 1) if i + 1 < len(chunks) else None
            cp.wait()
            s, c = chunks[i]
            b = bufs[i % 2]
            c0 = CS + c * CW
            tb8 = [tbs[s * 8 + r] for r in range(8)]

            def body(iv, carry, b=b, tb8=tb8, c0=c0):
                colv = ((jnp.zeros((LANES,), jnp.int32) + iv) * LANES
                        + lane_ids + c0)
                out = list(carry)
                for r in range(8):
                    xv = b[r, pl.ds(iv * LANES, LANES)]
                    hitf = jnp.where(colv == tb8[r], jnp.float32(1.0),
                                     jnp.float32(0.0))
                    out[r] = carry[r] + xv
                    out[8 + r] = carry[8 + r] + xv * hitf
                return tuple(out)

            res = lax.fori_loop(
                0, CW // LANES, body,
                tuple(jnp.zeros((LANES,), jnp.float32) for _ in range(16)))
            for r in range(8):
                validf = jnp.where(tb8[r] != 0, jnp.float32(1.0),
                                   jnp.float32(0.0))
                loss = loss + (jnp.float32(-SMOOTH) * res[r]
                               + jnp.float32(SMOOTH - CONF)
                               * res[8 + r]) * validf
            cp = nxt
        acc_v[...] = loss
        pltpu.sync_copy(acc_v, out_hbm.at[wid])

    return sc_fn


_sc_dense_cached = functools.lru_cache(maxsize=None)(_build_sc_dense)


def kernel(x, target):
    n, _ = x.shape
    t32 = target.astype(jnp.int32)
    sc_part = _sc_dense_cached(n)(x, t32)
    tc_part = _dense_sum(t32.reshape(n, 1), x)
    return tc_part[0, 0] + jnp.sum(sc_part)
